# baseline (speedup 1.0000x reference)
import functools

import jax
import jax.numpy as jnp
from jax import lax
from jax.experimental import pallas as pl
from jax.experimental.pallas import tpu as pltpu
from jax.experimental.pallas import tpu_sc as plsc

NUM_EMB = 1000000
DIM = 64
CHUNK = 128


@functools.lru_cache(maxsize=None)
def _build(n_total, n_workers, n_chunks):
    per_worker = n_chunks * CHUNK
    mesh = plsc.VectorSubcoreMesh(core_axis_name="c", subcore_axis_name="s")

    @functools.partial(
        pl.kernel,
        mesh=mesh,
        out_type=jax.ShapeDtypeStruct((n_total, DIM), jnp.float32),
        scratch_types=[
            pltpu.VMEM((n_chunks, CHUNK), jnp.int32),
            pltpu.VMEM((CHUNK, DIM), jnp.float32),
            pltpu.SemaphoreType.DMA,
        ],
        compiler_params=pltpu.CompilerParams(use_tc_tiling_on_sc=False),
    )
    def k(idx_hbm, table_hbm, out_hbm, idx_v, rows_v, sem):
        wid = lax.axis_index("s") * 2 + lax.axis_index("c")
        pltpu.sync_copy(idx_hbm.at[wid], idx_v)
        base = wid * per_worker

        def step(j, carry):
            pltpu.async_copy(table_hbm.at[idx_v.at[j]], rows_v, sem).wait()
            pltpu.sync_copy(rows_v, out_hbm.at[pl.ds(base + j * CHUNK, CHUNK)])
            return carry

        lax.fori_loop(0, n_chunks, step, 0)

    return k


def kernel(token_ids, weights):
    b, t = token_ids.shape
    n_total = b * t
    n_workers = 32
    assert n_total % (n_workers * CHUNK) == 0
    n_chunks = n_total // (n_workers * CHUNK)
    idx = token_ids.astype(jnp.int32).reshape(n_workers, n_chunks, CHUNK)
    out = _build(n_total, n_workers, n_chunks)(idx, weights)
    return out.reshape(b, t, weights.shape[1])


# speedup vs baseline: 1.1154x; 1.1154x over previous
import functools

import jax
import jax.numpy as jnp
from jax import lax
from jax.experimental import pallas as pl
from jax.experimental.pallas import tpu as pltpu
from jax.experimental.pallas import tpu_sc as plsc

DIM = 64
CHUNK = 128
K = 4
SUPER = CHUNK * K


@functools.lru_cache(maxsize=None)
def _build(n_total, n_workers, n_chunks):
    per_worker = n_chunks * CHUNK
    n_super = n_chunks // K
    mesh = plsc.VectorSubcoreMesh(core_axis_name="c", subcore_axis_name="s")

    @functools.partial(
        pl.kernel,
        mesh=mesh,
        out_type=jax.ShapeDtypeStruct((n_total, DIM), jnp.float32),
        scratch_types=[
            pltpu.VMEM((n_chunks, CHUNK), jnp.int32),
            pltpu.VMEM((SUPER, DIM), jnp.float32),
            pltpu.VMEM((SUPER, DIM), jnp.float32),
            pltpu.SemaphoreType.DMA,
            pltpu.SemaphoreType.DMA,
            pltpu.SemaphoreType.DMA,
            pltpu.SemaphoreType.DMA,
        ],
        compiler_params=pltpu.CompilerParams(use_tc_tiling_on_sc=False),
    )
    def k(idx_hbm, table_hbm, out_hbm, idx_v, rows_a, rows_b, sg_a, sg_b, sw_a,
          sw_b):
        wid = lax.axis_index("s") * 2 + lax.axis_index("c")
        pltpu.sync_copy(idx_hbm.at[wid], idx_v)
        base = wid * per_worker
        bufs = (rows_a, rows_b)
        gsems = (sg_a, sg_b)
        wsems = (sw_a, sw_b)

        def fire_gathers(s, b):
            for j in range(K):
                pltpu.async_copy(
                    table_hbm.at[idx_v.at[s * K + j]],
                    bufs[b].at[pl.ds(j * CHUNK, CHUNK)],
                    gsems[b],
                )

        def drain_gathers(b):
            for j in range(K):
                pltpu.make_async_copy(
                    table_hbm.at[idx_v.at[0]],
                    bufs[b].at[pl.ds(j * CHUNK, CHUNK)],
                    gsems[b],
                ).wait()

        def fire_write(s, b):
            pltpu.async_copy(
                bufs[b], out_hbm.at[pl.ds(base + s * SUPER, SUPER)], wsems[b]
            )

        def drain_write(b):
            pltpu.make_async_copy(
                bufs[b], out_hbm.at[pl.ds(0, SUPER)], wsems[b]
            ).wait()

        fire_gathers(0, 0)

        def step(s, b):
            @pl.when(s >= 1)
            def _():
                drain_write(1 - b)

            @pl.when(s + 1 < n_super)
            def _():
                fire_gathers(s + 1, 1 - b)

            drain_gathers(b)
            fire_write(s, b)

        def pair(i, carry):
            step(2 * i, 0)
            step(2 * i + 1, 1)
            return carry

        lax.fori_loop(0, n_super // 2, pair, 0)
        drain_write(1)

    return k


def kernel(token_ids, weights):
    b, t = token_ids.shape
    n_total = b * t
    n_workers = 32
    assert n_total % (n_workers * SUPER * 2) == 0
    n_chunks = n_total // (n_workers * CHUNK)
    idx = token_ids.astype(jnp.int32).reshape(n_workers, n_chunks, CHUNK)
    out = _build(n_total, n_workers, n_chunks)(idx, weights)
    return out.reshape(b, t, weights.shape[1])


# speedup vs baseline: 1.3590x; 1.2184x over previous
import functools

import jax
import jax.numpy as jnp
from jax import lax
from jax.experimental import pallas as pl
from jax.experimental.pallas import tpu as pltpu
from jax.experimental.pallas import tpu_sc as plsc

PDIM = 128
CHUNK = 128
K = 2
SUPER = CHUNK * K


@functools.lru_cache(maxsize=None)
def _build(n_total, n_workers, n_chunks):
    per_worker = n_chunks * CHUNK
    n_super = n_chunks // K
    mesh = plsc.VectorSubcoreMesh(core_axis_name="c", subcore_axis_name="s")

    @functools.partial(
        pl.kernel,
        mesh=mesh,
        out_type=jax.ShapeDtypeStruct((n_total, PDIM), jnp.float32),
        scratch_types=[
            pltpu.VMEM((n_chunks, CHUNK), jnp.int32),
            pltpu.VMEM((SUPER, PDIM), jnp.float32),
            pltpu.VMEM((SUPER, PDIM), jnp.float32),
            pltpu.SemaphoreType.DMA,
            pltpu.SemaphoreType.DMA,
            pltpu.SemaphoreType.DMA,
            pltpu.SemaphoreType.DMA,
        ],
    )
    def k(idx_hbm, table_hbm, out_hbm, idx_v, rows_a, rows_b, sg_a, sg_b, sw_a,
          sw_b):
        wid = lax.axis_index("s") * 2 + lax.axis_index("c")
        pltpu.sync_copy(idx_hbm.at[wid], idx_v)
        base = wid * per_worker
        bufs = (rows_a, rows_b)
        gsems = (sg_a, sg_b)
        wsems = (sw_a, sw_b)

        def fire_gathers(s, b):
            for j in range(K):
                pltpu.async_copy(
                    table_hbm.at[idx_v.at[s * K + j]],
                    bufs[b].at[pl.ds(j * CHUNK, CHUNK)],
                    gsems[b],
                )

        def drain_gathers(b):
            for j in range(K):
                pltpu.make_async_copy(
                    table_hbm.at[idx_v.at[0]],
                    bufs[b].at[pl.ds(j * CHUNK, CHUNK)],
                    gsems[b],
                ).wait()

        def fire_write(s, b):
            pltpu.async_copy(
                bufs[b], out_hbm.at[pl.ds(base + s * SUPER, SUPER)], wsems[b]
            )

        def drain_write(b):
            pltpu.make_async_copy(
                bufs[b], out_hbm.at[pl.ds(0, SUPER)], wsems[b]
            ).wait()

        fire_gathers(0, 0)

        def step(s, b):
            @pl.when(s >= 1)
            def _():
                drain_write(1 - b)

            @pl.when(s + 1 < n_super)
            def _():
                fire_gathers(s + 1, 1 - b)

            drain_gathers(b)
            fire_write(s, b)

        def pair(i, carry):
            step(2 * i, 0)
            step(2 * i + 1, 1)
            return carry

        lax.fori_loop(0, n_super // 2, pair, 0)
        drain_write(1)

    return k


def kernel(token_ids, weights):
    b, t = token_ids.shape
    dim = weights.shape[1]
    n_total = b * t
    n_workers = 32
    assert n_total % (n_workers * SUPER * 2) == 0
    n_chunks = n_total // (n_workers * CHUNK)
    idx = token_ids.astype(jnp.int32).reshape(n_workers, n_chunks, CHUNK)
    wpad = jnp.pad(weights, ((0, 0), (0, PDIM - dim)))
    out = _build(n_total, n_workers, n_chunks)(idx, wpad)
    return out[:, :dim].reshape(b, t, dim)


# speedup vs baseline: 1.3610x; 1.0015x over previous
import functools

import jax
import jax.numpy as jnp
from jax import lax
from jax.experimental import pallas as pl
from jax.experimental.pallas import tpu as pltpu
from jax.experimental.pallas import tpu_sc as plsc

PDIM = 128
CHUNK = 128
NBUF = 4


@functools.lru_cache(maxsize=None)
def _build(n_total, n_workers, n_chunks):
    per_worker = n_chunks * CHUNK
    mesh = plsc.VectorSubcoreMesh(core_axis_name="c", subcore_axis_name="s")

    @functools.partial(
        pl.kernel,
        mesh=mesh,
        out_type=jax.ShapeDtypeStruct((n_total, PDIM), jnp.float32),
        scratch_types=[
            pltpu.VMEM((n_chunks, CHUNK), jnp.int32),
        ]
        + [pltpu.VMEM((CHUNK, PDIM), jnp.float32)] * NBUF
        + [pltpu.SemaphoreType.DMA] * (2 * NBUF),
    )
    def k(idx_hbm, table_hbm, out_hbm, idx_v, *rest):
        bufs = rest[:NBUF]
        gsems = rest[NBUF : 2 * NBUF]
        wsems = rest[2 * NBUF :]
        wid = lax.axis_index("s") * 2 + lax.axis_index("c")
        pltpu.sync_copy(idx_hbm.at[wid], idx_v)
        base = wid * per_worker

        def fire_gather(t, r):
            pltpu.async_copy(table_hbm.at[idx_v.at[t]], bufs[r], gsems[r])

        def drain_gather(r):
            pltpu.make_async_copy(
                table_hbm.at[idx_v.at[0]], bufs[r], gsems[r]
            ).wait()

        def fire_write(t, r):
            pltpu.async_copy(
                bufs[r], out_hbm.at[pl.ds(base + t * CHUNK, CHUNK)], wsems[r]
            )

        def drain_write(r):
            pltpu.make_async_copy(
                bufs[r], out_hbm.at[pl.ds(0, CHUNK)], wsems[r]
            ).wait()

        fire_gather(0, 0)
        fire_gather(1, 1)

        def step(t, r):
            @pl.when(t >= 2)
            def _():
                drain_write((r - 2) % NBUF)

            @pl.when(t + 2 < n_chunks)
            def _():
                fire_gather(t + 2, (r + 2) % NBUF)

            drain_gather(r)
            fire_write(t, r)

        def quad(i, carry):
            for r in range(NBUF):
                step(NBUF * i + r, r)
            return carry

        lax.fori_loop(0, n_chunks // NBUF, quad, 0)
        drain_write(2)
        drain_write(3)

    return k


def kernel(token_ids, weights):
    b, t = token_ids.shape
    dim = weights.shape[1]
    n_total = b * t
    n_workers = 32
    n_chunks = n_total // (n_workers * CHUNK)
    assert n_total % (n_workers * CHUNK) == 0 and n_chunks % NBUF == 0
    idx = token_ids.astype(jnp.int32).reshape(n_workers, n_chunks, CHUNK)
    wpad = jnp.pad(weights, ((0, 0), (0, PDIM - dim)))
    out = _build(n_total, n_workers, n_chunks)(idx, wpad)
    return out[:, :dim].reshape(b, t, dim)
